# baseline (speedup 1.0000x reference)
import functools

import jax
import jax.numpy as jnp
from jax import lax
from jax.experimental import pallas as pl
from jax.experimental.pallas import tpu as pltpu
from jax.experimental.pallas import tpu_sc as plsc

N = 10000
E = 320000
D = 128
D_AUG = 144
D_MID = 90

NC = 2
NS = 16
L = 16
NW = NC * NS
EPW = E // NW
CH = 80
NCH = EPW // CH
UNR = 4
NITER = (NCH - 1) // UNR
WB = 1000
NWB = N // WB

_f32 = jnp.float32


def _sc_mesh():
    return plsc.VectorSubcoreMesh(core_axis_name="c", subcore_axis_name="s")

_SC_PARAMS = pltpu.CompilerParams(
    needs_layout_passes=False, use_tc_tiling_on_sc=False,
    disable_bounds_checks=True)


def _scale_rows(rows_v, b, w_v, k, width, unit=None):
    ngrp = width // L - (1 if unit is not None else 0)

    def grp(g, carry):
        base = g * L
        w16 = w_v[k, pl.ds(base, L)]
        for j in range(L):
            e = base + j
            wb = jnp.full((L,), w16[j], _f32)
            for q in range(ngrp):
                col = pl.ds(q * L, L)
                rows_v[b, e, col] = rows_v[b, e, col] * wb
            if unit is not None:
                rows_v[b, e, pl.ds(ngrp * L, L)] = wb * unit
        return carry
    lax.fori_loop(0, CH // L, grp, 0)


def _make_conv1():
    @functools.partial(
        pl.kernel,
        out_type=[
            jax.ShapeDtypeStruct((E,), _f32),
            jax.ShapeDtypeStruct((NC * N, D_AUG), _f32),
        ],
        mesh=_sc_mesh(),
        compiler_params=_SC_PARAMS,
        scratch_types=[
            pltpu.VMEM((N,), _f32),
            pltpu.VMEM((UNR, CH), jnp.int32),
            pltpu.VMEM((UNR, CH), jnp.int32),
            pltpu.VMEM((UNR, CH), _f32),
            pltpu.VMEM((UNR, CH), _f32),
            pltpu.VMEM((UNR, CH), _f32),
            pltpu.VMEM((2, CH, D_AUG), _f32),
            pltpu.VMEM_SHARED((N, D_AUG), _f32),
        ] + [pltpu.SemaphoreType.DMA] * 12,
    )
    def conv1(xa_hbm, srcr_hbm, dstr_hbm, et_hbm, ew_hbm, nt_hbm, znd_hbm,
              w_out, sx_out,
              nt_v, src_v, dst_v, et_v, ew_v, w_v, rows_v, agg_sh,
              sm0, sm1, sm2, sm3, sg0, sg1, ss0, ss1, sw0, sw1, sw2, sw3):
        sm = [sm0, sm1, sm2, sm3]
        sg = [sg0, sg1]
        ss = [ss0, ss1]
        sw = [sw0, sw1, sw2, sw3]
        cid = lax.axis_index("c")
        sid = lax.axis_index("s")
        wid = cid * NS + sid
        ebase = wid * EPW

        def meta_copies(c, k):
            base = ebase + c * CH
            return [
                (srcr_hbm.at[wid, pl.ds(c, 1)], src_v.at[pl.ds(k, 1)]),
                (dstr_hbm.at[wid, pl.ds(c, 1)], dst_v.at[pl.ds(k, 1)]),
                (et_hbm.at[pl.ds(base, CH)], et_v.at[k]),
                (ew_hbm.at[pl.ds(base, CH)], ew_v.at[k]),
            ]

        def issue_meta(c, k):
            for s, d in meta_copies(c, k):
                pltpu.async_copy(s, d, sm[k])

        def wait_meta(c, k):
            for s, d in meta_copies(c, k):
                pltpu.make_async_copy(s, d, sm[k]).wait()

        def issue_gather(k, b):
            pltpu.async_copy(xa_hbm.at[src_v.at[k]], rows_v.at[b], sg[b])

        def wait_gather(k, b):
            pltpu.make_async_copy(xa_hbm.at[src_v.at[k]], rows_v.at[b],
                                  sg[b]).wait()

        def issue_scatter(k, b):
            pltpu.async_copy(rows_v.at[b], agg_sh.at[dst_v.at[k]], ss[b],
                             add=True)

        def wait_scatter(b):
            pltpu.make_async_copy(rows_v.at[b], agg_sh.at[dst_v.at[0]],
                                  ss[b]).wait()

        def compute_w(c, k):
            for g in range(CH // L):
                s = pl.ds(g * L, L)
                d16 = dst_v[k, s]
                ntv = plsc.load_gather(nt_v, [d16])
                w_v[k, s] = jnp.exp(-jnp.abs(ntv - et_v[k, s])) * ew_v[k, s]

        def issue_w(c, k):
            pltpu.async_copy(w_v.at[k],
                             w_out.at[pl.ds(ebase + c * CH, CH)], sw[k])

        def wait_w(c, k):
            pltpu.make_async_copy(w_v.at[k],
                                  w_out.at[pl.ds(ebase + c * CH, CH)],
                                  sw[k]).wait()

        u16 = (lax.iota(jnp.int32, L) == 0).astype(_f32)
        pltpu.sync_copy(nt_hbm, nt_v)

        @pl.when(sid < NWB)
        def _():
            r0 = sid * WB
            pltpu.sync_copy(znd_hbm.at[pl.ds(r0, WB)], agg_sh.at[pl.ds(r0, WB)])
        plsc.subcore_barrier()

        for s, d in meta_copies(0, 0):
            pltpu.sync_copy(s, d)
        issue_meta(1, 1)
        issue_gather(0, 0)

        def body(cc, carry):
            for j in range(UNR):
                c = cc * UNR + j
                b2, n2 = j % 2, (j + 1) % 2
                n4, nn4 = (j + 1) % UNR, (j + 2) % UNR
                if j == UNR - 1:
                    @pl.when(cc < NITER - 1)
                    def _():
                        issue_meta(c + 2, nn4)
                else:
                    issue_meta(c + 2, nn4)
                @pl.when(cc > 0)
                def _():
                    wait_w(c - 4, j)
                compute_w(c, j)
                issue_w(c, j)
                wait_meta(c + 1, n4)
                if j == 0:
                    @pl.when(cc > 0)
                    def _():
                        wait_scatter(n2)
                else:
                    wait_scatter(n2)
                issue_gather(n4, n2)
                wait_gather(j, b2)
                _scale_rows(rows_v, b2, w_v, j, D_AUG, unit=u16)
                issue_scatter(j, b2)
            return carry

        lax.fori_loop(0, NITER, body, 0)

        ct = NCH - 1
        wait_w(ct - 4, 0)
        compute_w(ct, 0)
        issue_w(ct, 0)
        wait_gather(0, 0)
        _scale_rows(rows_v, 0, w_v, 0, D_AUG, unit=u16)
        pltpu.sync_copy(rows_v.at[0], agg_sh.at[dst_v.at[0]], add=True)
        wait_scatter(1)
        wait_w(ct - 3, 1)
        wait_w(ct - 2, 2)
        wait_w(ct - 1, 3)
        wait_w(ct, 0)
        plsc.subcore_barrier()

        @pl.when(sid < NWB)
        def _():
            r0 = sid * WB
            pltpu.sync_copy(agg_sh.at[pl.ds(r0, WB)],
                            sx_out.at[pl.ds(cid * N + r0, WB)])

    return conv1


def _make_conv2():
    @functools.partial(
        pl.kernel,
        out_type=[
            jax.ShapeDtypeStruct((NC * N, D), _f32),
        ],
        mesh=_sc_mesh(),
        compiler_params=_SC_PARAMS,
        scratch_types=[
            pltpu.VMEM((UNR, CH), jnp.int32),
            pltpu.VMEM((UNR, CH), jnp.int32),
            pltpu.VMEM((UNR, CH), jnp.int32),
            pltpu.VMEM((UNR, CH), _f32),
            pltpu.VMEM((UNR, CH, D), _f32),
            pltpu.VMEM_SHARED((N, D), _f32),
        ] + [pltpu.SemaphoreType.DMA] * 12,
    )
    def conv2(h_hbm, srcr_hbm, dstr_hbm, w_hbm, znd_hbm,
              s1_out,
              src_v, dst_v, dsts_v, w_v, rows_v, agg_sh,
              sm0, sm1, sm2, sm3, sg0, sg1, sg2, sg3, ss0, ss1, ss2, ss3):
        sm = [sm0, sm1, sm2, sm3]
        sg = [sg0, sg1, sg2, sg3]
        ss = [ss0, ss1, ss2, ss3]
        cid = lax.axis_index("c")
        sid = lax.axis_index("s")
        wid = cid * NS + sid
        ebase = wid * EPW

        def meta_copies(c, k):
            base = ebase + c * CH
            return [
                (srcr_hbm.at[wid, pl.ds(c, 1)], src_v.at[pl.ds(k, 1)]),
                (dstr_hbm.at[wid, pl.ds(c, 1)], dst_v.at[pl.ds(k, 1)]),
                (w_hbm.at[pl.ds(base, CH)], w_v.at[k]),
            ]

        def issue_meta(c, k):
            for s, d in meta_copies(c, k):
                pltpu.async_copy(s, d, sm[k])

        def wait_meta(c, k):
            for s, d in meta_copies(c, k):
                pltpu.make_async_copy(s, d, sm[k]).wait()

        def issue_gather(k, b):
            pltpu.async_copy(h_hbm.at[src_v.at[k]], rows_v.at[b], sg[b])

        def wait_gather(k, b):
            pltpu.make_async_copy(h_hbm.at[src_v.at[k]], rows_v.at[b],
                                  sg[b]).wait()

        def copy_dst(k):
            for g in range(CH // L):
                s = pl.ds(g * L, L)
                dsts_v[k, s] = dst_v[k, s]

        def issue_scatter(k):
            pltpu.async_copy(rows_v.at[k], agg_sh.at[dsts_v.at[k]], ss[k],
                             add=True)

        def wait_scatter(k):
            pltpu.make_async_copy(rows_v.at[k], agg_sh.at[dsts_v.at[k]],
                                  ss[k]).wait()

        @pl.when(sid < NWB)
        def _():
            r0 = sid * WB
            pltpu.sync_copy(znd_hbm.at[pl.ds(r0, WB), pl.ds(0, D)],
                            agg_sh.at[pl.ds(r0, WB)])
        plsc.subcore_barrier()

        for s, d in meta_copies(0, 0):
            pltpu.sync_copy(s, d)
        issue_meta(1, 1)
        issue_meta(2, 2)
        issue_gather(0, 0)
        wait_meta(1, 1)
        issue_gather(1, 1)

        def body(cc, carry):
            for j in range(UNR):
                n4, nn4 = (j + 1) % UNR, (j + 2) % UNR
                nnn4 = (j + 3) % UNR
                c = cc * UNR + j
                if j < 2:
                    issue_meta(c + 3, nnn4)
                else:
                    @pl.when(cc < NITER - 1)
                    def _():
                        issue_meta(c + 3, nnn4)
                if j == UNR - 1:
                    @pl.when(cc < NITER - 1)
                    def _():
                        wait_meta(c + 2, nn4)
                else:
                    wait_meta(c + 2, nn4)
                if j < 2:
                    @pl.when(cc > 0)
                    def _():
                        wait_scatter(nn4)
                else:
                    wait_scatter(nn4)
                if j == UNR - 1:
                    @pl.when(cc < NITER - 1)
                    def _():
                        issue_gather(nn4, nn4)
                else:
                    issue_gather(nn4, nn4)
                wait_gather(j, j)
                copy_dst(j)
                _scale_rows(rows_v, j, w_v, j, D)
                issue_scatter(j)
            return carry

        lax.fori_loop(0, NITER, body, 0)

        wait_gather(0, 0)
        copy_dst(0)
        _scale_rows(rows_v, 0, w_v, 0, D)
        pltpu.sync_copy(rows_v.at[0], agg_sh.at[dsts_v.at[0]], add=True)
        wait_scatter(2)
        wait_scatter(3)
        plsc.subcore_barrier()

        @pl.when(sid < NWB)
        def _():
            r0 = sid * WB
            pltpu.sync_copy(agg_sh.at[pl.ds(r0, WB)],
                            s1_out.at[pl.ds(cid * N + r0, WB)])

    return conv2


def _dense1_body(sx_ref, W1_ref, b1_ref, out1_ref, rinv_ref, t_ref):
    sxa = sx_ref[0] + sx_ref[1]
    sx = sxa[:, :D]
    deg = sxa[:, D:D + 1]
    rinv = 1.0 / (deg + 1e-6)
    t = deg * rinv
    h = jnp.dot(sx, W1_ref[...], preferred_element_type=_f32)
    out1_ref[...] = h * rinv + t * b1_ref[...].reshape(1, D)
    rinv_ref[...] = rinv
    t_ref[...] = t


def _dense2_body(s1_ref, rinv_ref, t_ref, W2_ref, b2_ref, g_ref, be_ref,
                 f1w_ref, f1b_ref, f2w_ref, f2b_ref, out_ref):
    s1 = s1_ref[0] + s1_ref[1]
    rinv = rinv_ref[...]
    t = t_ref[...]
    h = jnp.dot(s1, W2_ref[...], preferred_element_type=_f32) * rinv \
        + t * b2_ref[...].reshape(1, D)
    mu = jnp.mean(h, axis=0, keepdims=True)
    var = jnp.mean((h - mu) ** 2, axis=0, keepdims=True)
    h = (h - mu) * lax.rsqrt(var + 1e-5) * g_ref[...].reshape(1, D) \
        + be_ref[...].reshape(1, D)
    h = jnp.where(h >= 0, h, 0.01 * h)
    h = jnp.dot(h, f1w_ref[...], preferred_element_type=_f32) \
        + f1b_ref[...].reshape(1, D_MID)
    h = jnp.where(h >= 0, h, 0.01 * h)
    out_ref[...] = jnp.dot(h, f2w_ref[...], preferred_element_type=_f32) \
        + f2b_ref[...].reshape(1, D)


def kernel(x, edge_index, edge_time, node_time, edge_weight,
           W1, b1, W2, b2, bn_gamma, bn_beta, fc1_W, fc1_b, fc2_W, fc2_b):
    srcr = edge_index[0].reshape(NW, NCH, CH)
    dstr = edge_index[1].reshape(NW, NCH, CH)
    xa = jnp.concatenate(
        [x, jnp.ones((N, 1), _f32), jnp.zeros((N, D_AUG - D - 1), _f32)],
        axis=1)
    znd = jnp.zeros((N, D_AUG), _f32)

    conv1 = _make_conv1()
    w, sx_p = conv1(xa, srcr, dstr, edge_time, edge_weight, node_time, znd)

    out1, rinv, t = pl.pallas_call(
        _dense1_body,
        out_shape=[
            jax.ShapeDtypeStruct((N, D), _f32),
            jax.ShapeDtypeStruct((N, 1), _f32),
            jax.ShapeDtypeStruct((N, 1), _f32),
        ],
    )(sx_p.reshape(NC, N, D_AUG), W1, b1)

    conv2 = _make_conv2()
    (s1_p,) = conv2(out1, srcr, dstr, w, znd)

    out = pl.pallas_call(
        _dense2_body,
        out_shape=jax.ShapeDtypeStruct((N, D), _f32),
    )(s1_p.reshape(NC, N, D), rinv, t, W2, b2, bn_gamma, bn_beta,
      fc1_W, fc1_b, fc2_W, fc2_b)
    return out


# speedup vs baseline: 1.0027x; 1.0027x over previous
import functools

import jax
import jax.numpy as jnp
from jax import lax
from jax.experimental import pallas as pl
from jax.experimental.pallas import tpu as pltpu
from jax.experimental.pallas import tpu_sc as plsc

N = 10000
E = 320000
D = 128
D_AUG = 144
D_MID = 90

NC = 2
NS = 16
L = 16
NW = NC * NS
EPW = E // NW
CH = 80
NCH = EPW // CH
UNR = 4
NITER = (NCH - 1) // UNR
WB = 1000
NWB = N // WB

_f32 = jnp.float32


def _sc_mesh():
    return plsc.VectorSubcoreMesh(core_axis_name="c", subcore_axis_name="s")

_SC_PARAMS = pltpu.CompilerParams(
    needs_layout_passes=False, use_tc_tiling_on_sc=False)


def _scale_rows(rows_v, b, w_v, k, width, unit=None):
    ngrp = width // L - (1 if unit is not None else 0)

    def grp(g, carry):
        base = g * L
        w16 = w_v[k, pl.ds(base, L)]
        for j in range(L):
            e = base + j
            wb = jnp.full((L,), w16[j], _f32)
            for q in range(ngrp):
                col = pl.ds(q * L, L)
                rows_v[b, e, col] = rows_v[b, e, col] * wb
            if unit is not None:
                rows_v[b, e, pl.ds(ngrp * L, L)] = wb * unit
        return carry
    lax.fori_loop(0, CH // L, grp, 0)


def _make_conv1():
    @functools.partial(
        pl.kernel,
        out_type=[
            jax.ShapeDtypeStruct((E,), _f32),
            jax.ShapeDtypeStruct((NC * N, D_AUG), _f32),
        ],
        mesh=_sc_mesh(),
        compiler_params=_SC_PARAMS,
        scratch_types=[
            pltpu.VMEM((N,), _f32),
            pltpu.VMEM((UNR, CH), jnp.int32),
            pltpu.VMEM((UNR, CH), jnp.int32),
            pltpu.VMEM((UNR, CH), _f32),
            pltpu.VMEM((UNR, CH), _f32),
            pltpu.VMEM((UNR, CH), _f32),
            pltpu.VMEM((2, CH, D_AUG), _f32),
            pltpu.VMEM_SHARED((N, D_AUG), _f32),
        ] + [pltpu.SemaphoreType.DMA] * 12,
    )
    def conv1(xa_hbm, srcr_hbm, dstr_hbm, et_hbm, ew_hbm, nt_hbm, znd_hbm,
              w_out, sx_out,
              nt_v, src_v, dst_v, et_v, ew_v, w_v, rows_v, agg_sh,
              sm0, sm1, sm2, sm3, sg0, sg1, ss0, ss1, sw0, sw1, sw2, sw3):
        sm = [sm0, sm1, sm2, sm3]
        sg = [sg0, sg1]
        ss = [ss0, ss1]
        sw = [sw0, sw1, sw2, sw3]
        cid = lax.axis_index("c")
        sid = lax.axis_index("s")
        wid = cid * NS + sid
        ebase = wid * EPW

        def meta_copies(c, k):
            base = ebase + c * CH
            return [
                (srcr_hbm.at[wid, pl.ds(c, 1)], src_v.at[pl.ds(k, 1)]),
                (dstr_hbm.at[wid, pl.ds(c, 1)], dst_v.at[pl.ds(k, 1)]),
                (et_hbm.at[pl.ds(base, CH)], et_v.at[k]),
                (ew_hbm.at[pl.ds(base, CH)], ew_v.at[k]),
            ]

        def issue_meta(c, k):
            for s, d in meta_copies(c, k):
                pltpu.async_copy(s, d, sm[k])

        def wait_meta(c, k):
            for s, d in meta_copies(c, k):
                pltpu.make_async_copy(s, d, sm[k]).wait()

        def issue_gather(k, b):
            pltpu.async_copy(xa_hbm.at[src_v.at[k]], rows_v.at[b], sg[b])

        def wait_gather(k, b):
            pltpu.make_async_copy(xa_hbm.at[src_v.at[k]], rows_v.at[b],
                                  sg[b]).wait()

        def issue_scatter(k, b):
            pltpu.async_copy(rows_v.at[b], agg_sh.at[dst_v.at[k]], ss[b],
                             add=True)

        def wait_scatter(b):
            pltpu.make_async_copy(rows_v.at[b], agg_sh.at[dst_v.at[0]],
                                  ss[b]).wait()

        def compute_w(c, k):
            for g in range(CH // L):
                s = pl.ds(g * L, L)
                d16 = dst_v[k, s]
                ntv = plsc.load_gather(nt_v, [d16])
                w_v[k, s] = jnp.exp(-jnp.abs(ntv - et_v[k, s])) * ew_v[k, s]

        def issue_w(c, k):
            pltpu.async_copy(w_v.at[k],
                             w_out.at[pl.ds(ebase + c * CH, CH)], sw[k])

        def wait_w(c, k):
            pltpu.make_async_copy(w_v.at[k],
                                  w_out.at[pl.ds(ebase + c * CH, CH)],
                                  sw[k]).wait()

        u16 = (lax.iota(jnp.int32, L) == 0).astype(_f32)
        pltpu.sync_copy(nt_hbm, nt_v)

        @pl.when(sid < NWB)
        def _():
            r0 = sid * WB
            pltpu.sync_copy(znd_hbm.at[pl.ds(r0, WB)], agg_sh.at[pl.ds(r0, WB)])
        plsc.subcore_barrier()

        for s, d in meta_copies(0, 0):
            pltpu.sync_copy(s, d)
        issue_meta(1, 1)
        issue_gather(0, 0)

        def body(cc, carry):
            for j in range(UNR):
                c = cc * UNR + j
                b2, n2 = j % 2, (j + 1) % 2
                n4, nn4 = (j + 1) % UNR, (j + 2) % UNR
                if j == UNR - 1:
                    @pl.when(cc < NITER - 1)
                    def _():
                        issue_meta(c + 2, nn4)
                else:
                    issue_meta(c + 2, nn4)
                @pl.when(cc > 0)
                def _():
                    wait_w(c - 4, j)
                compute_w(c, j)
                issue_w(c, j)
                wait_meta(c + 1, n4)
                if j == 0:
                    @pl.when(cc > 0)
                    def _():
                        wait_scatter(n2)
                else:
                    wait_scatter(n2)
                issue_gather(n4, n2)
                wait_gather(j, b2)
                _scale_rows(rows_v, b2, w_v, j, D_AUG, unit=u16)
                issue_scatter(j, b2)
            return carry

        lax.fori_loop(0, NITER, body, 0)

        ct = NCH - 1
        wait_w(ct - 4, 0)
        compute_w(ct, 0)
        issue_w(ct, 0)
        wait_gather(0, 0)
        _scale_rows(rows_v, 0, w_v, 0, D_AUG, unit=u16)
        pltpu.sync_copy(rows_v.at[0], agg_sh.at[dst_v.at[0]], add=True)
        wait_scatter(1)
        wait_w(ct - 3, 1)
        wait_w(ct - 2, 2)
        wait_w(ct - 1, 3)
        wait_w(ct, 0)
        plsc.subcore_barrier()

        @pl.when(sid < NWB)
        def _():
            r0 = sid * WB
            pltpu.sync_copy(agg_sh.at[pl.ds(r0, WB)],
                            sx_out.at[pl.ds(cid * N + r0, WB)])

    return conv1


def _make_conv2():
    @functools.partial(
        pl.kernel,
        out_type=[
            jax.ShapeDtypeStruct((NC * N, D), _f32),
        ],
        mesh=_sc_mesh(),
        compiler_params=_SC_PARAMS,
        scratch_types=[
            pltpu.VMEM((UNR, CH), jnp.int32),
            pltpu.VMEM((UNR, CH), jnp.int32),
            pltpu.VMEM((UNR, CH), jnp.int32),
            pltpu.VMEM((UNR, CH), _f32),
            pltpu.VMEM((UNR, CH, D), _f32),
            pltpu.VMEM_SHARED((N, D), _f32),
        ] + [pltpu.SemaphoreType.DMA] * 12,
    )
    def conv2(h_hbm, srcr_hbm, dstr_hbm, w_hbm, znd_hbm,
              s1_out,
              src_v, dst_v, dsts_v, w_v, rows_v, agg_sh,
              sm0, sm1, sm2, sm3, sg0, sg1, sg2, sg3, ss0, ss1, ss2, ss3):
        sm = [sm0, sm1, sm2, sm3]
        sg = [sg0, sg1, sg2, sg3]
        ss = [ss0, ss1, ss2, ss3]
        cid = lax.axis_index("c")
        sid = lax.axis_index("s")
        wid = cid * NS + sid
        ebase = wid * EPW

        def meta_copies(c, k):
            base = ebase + c * CH
            return [
                (srcr_hbm.at[wid, pl.ds(c, 1)], src_v.at[pl.ds(k, 1)]),
                (dstr_hbm.at[wid, pl.ds(c, 1)], dst_v.at[pl.ds(k, 1)]),
                (w_hbm.at[pl.ds(base, CH)], w_v.at[k]),
            ]

        def issue_meta(c, k):
            for s, d in meta_copies(c, k):
                pltpu.async_copy(s, d, sm[k])

        def wait_meta(c, k):
            for s, d in meta_copies(c, k):
                pltpu.make_async_copy(s, d, sm[k]).wait()

        def issue_gather(k, b):
            pltpu.async_copy(h_hbm.at[src_v.at[k]], rows_v.at[b], sg[b])

        def wait_gather(k, b):
            pltpu.make_async_copy(h_hbm.at[src_v.at[k]], rows_v.at[b],
                                  sg[b]).wait()

        def copy_dst(k):
            for g in range(CH // L):
                s = pl.ds(g * L, L)
                dsts_v[k, s] = dst_v[k, s]

        def issue_scatter(k):
            pltpu.async_copy(rows_v.at[k], agg_sh.at[dsts_v.at[k]], ss[k],
                             add=True)

        def wait_scatter(k):
            pltpu.make_async_copy(rows_v.at[k], agg_sh.at[dsts_v.at[k]],
                                  ss[k]).wait()

        @pl.when(sid < NWB)
        def _():
            r0 = sid * WB
            pltpu.sync_copy(znd_hbm.at[pl.ds(r0, WB), pl.ds(0, D)],
                            agg_sh.at[pl.ds(r0, WB)])
        plsc.subcore_barrier()

        for s, d in meta_copies(0, 0):
            pltpu.sync_copy(s, d)
        issue_meta(1, 1)
        issue_meta(2, 2)
        issue_gather(0, 0)
        wait_meta(1, 1)
        issue_gather(1, 1)

        def body(cc, carry):
            for j in range(UNR):
                n4, nn4 = (j + 1) % UNR, (j + 2) % UNR
                nnn4 = (j + 3) % UNR
                c = cc * UNR + j
                if j < 2:
                    issue_meta(c + 3, nnn4)
                else:
                    @pl.when(cc < NITER - 1)
                    def _():
                        issue_meta(c + 3, nnn4)
                if j == UNR - 1:
                    @pl.when(cc < NITER - 1)
                    def _():
                        wait_meta(c + 2, nn4)
                else:
                    wait_meta(c + 2, nn4)
                if j < 2:
                    @pl.when(cc > 0)
                    def _():
                        wait_scatter(nn4)
                else:
                    wait_scatter(nn4)
                if j == UNR - 1:
                    @pl.when(cc < NITER - 1)
                    def _():
                        issue_gather(nn4, nn4)
                else:
                    issue_gather(nn4, nn4)
                wait_gather(j, j)
                copy_dst(j)
                _scale_rows(rows_v, j, w_v, j, D)
                issue_scatter(j)
            return carry

        lax.fori_loop(0, NITER, body, 0)

        wait_gather(0, 0)
        copy_dst(0)
        _scale_rows(rows_v, 0, w_v, 0, D)
        pltpu.sync_copy(rows_v.at[0], agg_sh.at[dsts_v.at[0]], add=True)
        wait_scatter(2)
        wait_scatter(3)
        plsc.subcore_barrier()

        @pl.when(sid < NWB)
        def _():
            r0 = sid * WB
            pltpu.sync_copy(agg_sh.at[pl.ds(r0, WB)],
                            s1_out.at[pl.ds(cid * N + r0, WB)])

    return conv2


def _dense1_body(sx_ref, W1_ref, b1_ref, out1_ref, rinv_ref, t_ref):
    sxa = sx_ref[0] + sx_ref[1]
    sx = sxa[:, :D]
    deg = sxa[:, D:D + 1]
    rinv = 1.0 / (deg + 1e-6)
    t = deg * rinv
    h = jnp.dot(sx, W1_ref[...], preferred_element_type=_f32)
    out1_ref[...] = h * rinv + t * b1_ref[...].reshape(1, D)
    rinv_ref[...] = rinv
    t_ref[...] = t


def _dense2_body(s1_ref, rinv_ref, t_ref, W2_ref, b2_ref, g_ref, be_ref,
                 f1w_ref, f1b_ref, f2w_ref, f2b_ref, out_ref):
    s1 = s1_ref[0] + s1_ref[1]
    rinv = rinv_ref[...]
    t = t_ref[...]
    h = jnp.dot(s1, W2_ref[...], preferred_element_type=_f32) * rinv \
        + t * b2_ref[...].reshape(1, D)
    mu = jnp.mean(h, axis=0, keepdims=True)
    var = jnp.mean((h - mu) ** 2, axis=0, keepdims=True)
    h = (h - mu) * lax.rsqrt(var + 1e-5) * g_ref[...].reshape(1, D) \
        + be_ref[...].reshape(1, D)
    h = jnp.where(h >= 0, h, 0.01 * h)
    h = jnp.dot(h, f1w_ref[...], preferred_element_type=_f32) \
        + f1b_ref[...].reshape(1, D_MID)
    h = jnp.where(h >= 0, h, 0.01 * h)
    out_ref[...] = jnp.dot(h, f2w_ref[...], preferred_element_type=_f32) \
        + f2b_ref[...].reshape(1, D)


def kernel(x, edge_index, edge_time, node_time, edge_weight,
           W1, b1, W2, b2, bn_gamma, bn_beta, fc1_W, fc1_b, fc2_W, fc2_b):
    srcr = edge_index[0].reshape(NW, NCH, CH)
    dstr = edge_index[1].reshape(NW, NCH, CH)
    xa = jnp.concatenate(
        [x, jnp.ones((N, 1), _f32), jnp.zeros((N, D_AUG - D - 1), _f32)],
        axis=1)
    znd = jnp.zeros((N, D_AUG), _f32)

    conv1 = _make_conv1()
    w, sx_p = conv1(xa, srcr, dstr, edge_time, edge_weight, node_time, znd)

    out1, rinv, t = pl.pallas_call(
        _dense1_body,
        out_shape=[
            jax.ShapeDtypeStruct((N, D), _f32),
            jax.ShapeDtypeStruct((N, 1), _f32),
            jax.ShapeDtypeStruct((N, 1), _f32),
        ],
    )(sx_p.reshape(NC, N, D_AUG), W1, b1)

    conv2 = _make_conv2()
    (s1_p,) = conv2(out1, srcr, dstr, w, znd)

    out = pl.pallas_call(
        _dense2_body,
        out_shape=jax.ShapeDtypeStruct((N, D), _f32),
    )(s1_p.reshape(NC, N, D), rinv, t, W2, b2, bn_gamma, bn_beta,
      fc1_W, fc1_b, fc2_W, fc2_b)
    return out
